# baseline (speedup 1.0000x reference)
import functools

import jax
import jax.numpy as jnp
from jax import lax
from jax.experimental import pallas as pl
from jax.experimental.pallas import tpu as pltpu
from jax.experimental.pallas import tpu_sc as plsc

N = 50000
D = 64
NL = 4
L = 2
E0 = 200000
PAD = 704
E0P = E0 + PAD
EGP = NL * E0P
CHUNK = 896
NSTREAM = CHUNK // 128
DCHUNK = 256
DNS = DCHUNK // 128
HALF = 25000
TRASH = HALF
ACC_ROWS = HALF + 8
ZCH = 1000
NP = 51200
NDEN = 57344

_mesh = plsc.VectorSubcoreMesh(core_axis_name="c", subcore_axis_name="s")
f32 = jnp.float32
i32 = jnp.int32


def _proj_body(x_ref, w_ref, ad_ref, h4_ref, pd_ref):
    xb = x_ref[...]
    h = jnp.dot(xb, w_ref[0], preferred_element_type=f32)
    h4_ref[...] = h
    pd_ref[0, 0] = jnp.sum(h * ad_ref[0, 0][None, :], axis=1)


def _stage_a(x, w_agg, a_dst):
    bk = 2048
    nbn = NP // bk
    return pl.pallas_call(
        _proj_body,
        grid=(NL, nbn),
        in_specs=[
            pl.BlockSpec((bk, D), lambda c, i: (i, 0)),
            pl.BlockSpec((1, D, D), lambda c, i: (c, 0, 0)),
            pl.BlockSpec((1, 1, D), lambda c, i: (c, 0, 0)),
        ],
        out_specs=[
            pl.BlockSpec((bk, D), lambda c, i: (c * nbn + i, 0)),
            pl.BlockSpec((1, 1, bk), lambda c, i: (c, 0, i)),
        ],
        out_shape=[
            jax.ShapeDtypeStruct((NL * NP, D), f32),
            jax.ShapeDtypeStruct((NL, 1, NP), f32),
        ],
    )(x, w_agg, a_dst.reshape(NL, 1, D))


@functools.partial(
    pl.kernel,
    out_type=(
        jax.ShapeDtypeStruct((EGP, D), f32),
        jax.ShapeDtypeStruct((EGP,), f32),
    ),
    mesh=_mesh,
    scratch_types=[
        pltpu.VMEM((CHUNK,), i32),
        pltpu.VMEM((CHUNK,), i32),
        pltpu.VMEM((CHUNK, D), f32),
        pltpu.VMEM((CHUNK,), f32),
        pltpu.SemaphoreType.DMA,
        pltpu.SemaphoreType.DMA,
    ],
    compiler_params=pltpu.CompilerParams(use_tc_tiling_on_sc=False),
)
def _gather_kernel(h4f, pdt, srcf, dstg, rows_out, pdg_out,
                   sidx, didx, rbuf, pbuf, sem_r, sem_p):
    wid = lax.axis_index("s") * 2 + lax.axis_index("c")
    per_tile = EGP // 32

    def chunk(k, carry):
        base = wid * per_tile + k * CHUNK
        pltpu.sync_copy(srcf.at[pl.ds(base, CHUNK)], sidx)
        pltpu.sync_copy(dstg.at[pl.ds(base, CHUNK)], didx)
        cps = []
        for j in range(NSTREAM):
            sl = pl.ds(j * 128, 128)
            cps.append(pltpu.async_copy(h4f.at[sidx.at[sl]], rbuf.at[sl], sem_r))
            cps.append(pltpu.async_copy(pdt.at[didx.at[sl]], pbuf.at[sl], sem_p))
        for cp in cps:
            cp.wait()
        pltpu.sync_copy(rbuf, rows_out.at[pl.ds(base, CHUNK)])
        pltpu.sync_copy(pbuf, pdg_out.at[pl.ds(base, CHUNK)])
        return carry

    lax.fori_loop(0, per_tile // CHUNK, chunk, 0)


def _scale_body(rows_ref, pdg_ref, as_ref, scaled_ref, w_ref):
    rows = rows_ref[...]
    ps = jnp.dot(rows, as_ref[0, 0].reshape(D, 1), preferred_element_type=f32,
                 precision=jax.lax.Precision.HIGHEST)
    e = ps[:, 0] + pdg_ref[...].reshape(-1)
    e = jnp.where(e > 0, e, 0.2 * e)
    w = jnp.exp(e)
    scaled_ref[...] = rows * w[:, None]
    w_ref[...] = w.reshape(w_ref.shape)


def _stage_c(rows, pdg2, a_src):
    bk = 4096
    nb = E0P // bk
    return pl.pallas_call(
        _scale_body,
        grid=(NL, nb),
        in_specs=[
            pl.BlockSpec((bk, D), lambda c, i: (c * nb + i, 0)),
            pl.BlockSpec((bk // 128, 128), lambda c, i: (c * nb + i, 0)),
            pl.BlockSpec((1, 1, D), lambda c, i: (c, 0, 0)),
        ],
        out_specs=[
            pl.BlockSpec((bk, D), lambda c, i: (c * nb + i, 0)),
            pl.BlockSpec((bk // 128, 128), lambda c, i: (c * nb + i, 0)),
        ],
        out_shape=[
            jax.ShapeDtypeStruct((EGP, D), f32),
            jax.ShapeDtypeStruct((EGP // 128, 128), f32),
        ],
    )(rows, pdg2, a_src.reshape(NL, 1, D))


@functools.partial(
    pl.kernel,
    out_type=(
        jax.ShapeDtypeStruct((NL, N, D), f32),
        jax.ShapeDtypeStruct((NL, NDEN), f32),
    ),
    mesh=_mesh,
    scratch_types=[
        pltpu.VMEM_SHARED((ACC_ROWS, D), f32),
        pltpu.VMEM_SHARED((ACC_ROWS,), f32),
        pltpu.VMEM((DCHUNK, D), f32),
        pltpu.VMEM((DCHUNK,), f32),
        pltpu.VMEM((DCHUNK,), i32),
        pltpu.VMEM((DNS, 128), i32),
        pltpu.SemaphoreType.DMA,
    ],
    compiler_params=pltpu.CompilerParams(use_tc_tiling_on_sc=False),
)
def _scatter_kernel(scaledf, wf, dstn, zrow, zden, acc_out, den_out,
                    shacc, shden, sbuf, wbuf, dbuf, idx2, sem):
    cid = lax.axis_index("c")
    sid = lax.axis_index("s")
    off = cid * HALF
    per_tile = E0P // 16

    for c in range(NL):
        def zchunk(kk, carry):
            ch = kk * 16 + sid

            @pl.when(ch < HALF // ZCH)
            def _():
                pltpu.sync_copy(zrow, shacc.at[pl.ds(ch * ZCH, ZCH)])
                pltpu.sync_copy(zden, shden.at[pl.ds(ch * ZCH, ZCH)])
            return carry

        lax.fori_loop(0, 2, zchunk, 0)

        @pl.when(sid == 0)
        def _():
            pltpu.sync_copy(zrow.at[pl.ds(0, 8)], shacc.at[pl.ds(HALF, 8)])
            pltpu.sync_copy(zden.at[pl.ds(0, 8)], shden.at[pl.ds(HALF, 8)])
        plsc.subcore_barrier()

        def echunk(k, carry):
            base = c * E0P + sid * per_tile + k * DCHUNK
            l1 = pltpu.async_copy(dstn.at[pl.ds(base, DCHUNK)], dbuf, sem)
            l2 = pltpu.async_copy(wf.at[pl.ds(base, DCHUNK)], wbuf, sem)
            l3 = pltpu.async_copy(scaledf.at[pl.ds(base, DCHUNK)], sbuf, sem)
            l1.wait()
            l2.wait()
            l3.wait()
            for j in range(DNS):
                for q in range(8):
                    dv = dbuf[pl.ds(j * 128 + q * 16, 16)]
                    lc = dv - off
                    ok = (lc >= 0) & (lc < HALF)
                    idx2[j, pl.ds(q * 16, 16)] = jnp.where(ok, lc, TRASH)
            for j in range(DNS):
                sl = pl.ds(j * 128, 128)
                pltpu.sync_copy(sbuf.at[sl], shacc.at[idx2.at[j]], add=True)
                pltpu.sync_copy(wbuf.at[sl], shden.at[idx2.at[j]], add=True)
            return carry

        lax.fori_loop(0, per_tile // DCHUNK, echunk, 0)
        plsc.subcore_barrier()

        def ochunk(kk, carry):
            ch = kk * 16 + sid

            @pl.when(ch < HALF // ZCH)
            def _():
                row = off + ch * ZCH
                pltpu.sync_copy(shacc.at[pl.ds(ch * ZCH, ZCH)],
                                acc_out.at[c, pl.ds(row, ZCH)])
                pltpu.sync_copy(shden.at[pl.ds(ch * ZCH, ZCH)],
                                den_out.at[c, pl.ds(row, ZCH)])
            return carry

        lax.fori_loop(0, 2, ochunk, 0)
        plsc.subcore_barrier()


def _mlp_body(x_ref, h0_ref, h1_ref, h2_ref, h3_ref, pd_ref, acc_ref, den_ref,
              as_ref, bagg_ref, w1_ref, b1_ref, w2_ref, b2_ref, y_ref):
    xb = x_ref[...]
    h_refs = (h0_ref, h1_ref, h2_ref, h3_ref)
    t = jnp.dot(xb, w1_ref[0:D, :], preferred_element_type=f32)
    for c in range(NL):
        hb = h_refs[c][...]
        ps = jnp.sum(hb * as_ref[c][None, :], axis=1)
        e = ps + pd_ref[c, 0]
        e = jnp.where(e > 0, e, 0.2 * e)
        wl = jnp.exp(e)
        den = den_ref[c].reshape(-1)
        inv = (1.0 / (den + wl + 1e-16))[:, None]
        out = (acc_ref[c] + wl[:, None] * hb) * inv + bagg_ref[c][None, :]
        t = t + jnp.dot(out, w1_ref[D * (c + 1):D * (c + 2), :],
                        preferred_element_type=f32)
    t = jnp.tanh(t + b1_ref[0][None, :])
    y_ref[...] = jnp.dot(t, w2_ref[...], preferred_element_type=f32) + b2_ref[0][None, :]


def _stage_e(x, h4f, pd3, acc, den, a_src, b_agg, w1, b1, w2, b2):
    bk = 2048
    nb = (N + bk - 1) // bk
    nbn = NP // bk
    den3 = den.reshape(NL, NDEN // 128, 128)
    hspecs = [pl.BlockSpec((bk, D), (lambda i, c=c: (c * nbn + i, 0)))
              for c in range(NL)]
    return pl.pallas_call(
        _mlp_body,
        grid=(nb,),
        in_specs=[
            pl.BlockSpec((bk, D), lambda i: (i, 0)),
            *hspecs,
            pl.BlockSpec((NL, 1, bk), lambda i: (0, 0, i)),
            pl.BlockSpec((NL, bk, D), lambda i: (0, i, 0)),
            pl.BlockSpec((NL, bk // 128, 128), lambda i: (0, i, 0)),
            pl.BlockSpec((NL, D), lambda i: (0, 0)),
            pl.BlockSpec((NL, D), lambda i: (0, 0)),
            pl.BlockSpec(((NL + 1) * D, D), lambda i: (0, 0)),
            pl.BlockSpec((1, D), lambda i: (0, 0)),
            pl.BlockSpec((D, D), lambda i: (0, 0)),
            pl.BlockSpec((1, D), lambda i: (0, 0)),
        ],
        out_specs=pl.BlockSpec((bk, D), lambda i: (i, 0)),
        out_shape=jax.ShapeDtypeStruct((N, D), f32),
    )(x, h4f, h4f, h4f, h4f, pd3, acc, den3, a_src, b_agg, w1,
      b1.reshape(1, D), w2, b2.reshape(1, D))


def kernel(x, edge_index, W_agg, a_src, a_dst, b_agg, W1, b1, W2, b2):
    src = edge_index[0].reshape(NL, E0).astype(i32)
    dst = edge_index[1].reshape(NL, E0).astype(i32)
    offs = (jnp.arange(NL, dtype=i32) * NP)[:, None]
    srcf = jnp.concatenate([src + offs, jnp.zeros((NL, PAD), i32)], 1).reshape(-1)
    dstg = jnp.concatenate([dst + offs, jnp.zeros((NL, PAD), i32)], 1).reshape(-1)
    dstn = jnp.concatenate([dst, jnp.full((NL, PAD), N, i32)], 1).reshape(-1)
    zrow = jnp.zeros((ZCH, D), f32)
    zden = jnp.zeros((ZCH,), f32)

    h = x
    for l in range(L):
        h4f, pd3 = _stage_a(h, W_agg[l], a_dst[l])
        rows, pdg = _gather_kernel(h4f, pd3.reshape(-1), srcf, dstg)
        scaled, w2d = _stage_c(rows, pdg.reshape(EGP // 128, 128), a_src[l])
        acc, den = _scatter_kernel(scaled, w2d.reshape(-1), dstn, zrow, zden)
        h = _stage_e(h, h4f, pd3, acc, den, a_src[l], b_agg[l],
                     W1[l], b1[l], W2[l], b2[l])
    return h


# speedup vs baseline: 1.0276x; 1.0276x over previous
import functools

import jax
import jax.numpy as jnp
from jax import lax
from jax.experimental import pallas as pl
from jax.experimental.pallas import tpu as pltpu
from jax.experimental.pallas import tpu_sc as plsc

N = 50000
D = 64
NL = 4
L = 2
E0 = 200000
PAD = 704
E0P = E0 + PAD
EGP = NL * E0P
CHUNK = 896
NSTREAM = CHUNK // 128
DCHUNK = 256
DNS = DCHUNK // 128
HALF = 25000
TRASH = HALF
ACC_ROWS = HALF + 8
ZCH = 1000
NP = 51200
NDEN = 57344

_mesh = plsc.VectorSubcoreMesh(core_axis_name="c", subcore_axis_name="s")
f32 = jnp.float32
i32 = jnp.int32


def _proj_body(x_ref, w_ref, ad_ref, h4_ref, pd_ref):
    xb = x_ref[...]
    cols = []
    for c in range(NL):
        h = jnp.dot(xb, w_ref[c], preferred_element_type=f32)
        h4_ref[c] = h
        cols.append(jnp.sum(h * ad_ref[c][None, :], axis=1))
    pd_ref[...] = jnp.stack(cols, axis=1)


def _stage_a(x, w_agg, a_dst):
    bk = 2000
    return pl.pallas_call(
        _proj_body,
        grid=(N // bk,),
        in_specs=[
            pl.BlockSpec((bk, D), lambda i: (i, 0)),
            pl.BlockSpec((NL, D, D), lambda i: (0, 0, 0)),
            pl.BlockSpec((NL, D), lambda i: (0, 0)),
        ],
        out_specs=[
            pl.BlockSpec((NL, bk, D), lambda i: (0, i, 0)),
            pl.BlockSpec((bk, NL), lambda i: (i, 0)),
        ],
        out_shape=[
            jax.ShapeDtypeStruct((NL, N, D), f32),
            jax.ShapeDtypeStruct((N, NL), f32),
        ],
    )(x, w_agg, a_dst)


@functools.partial(
    pl.kernel,
    out_type=(
        jax.ShapeDtypeStruct((EGP, D), f32),
        jax.ShapeDtypeStruct((EGP,), f32),
    ),
    mesh=_mesh,
    scratch_types=[
        pltpu.VMEM((CHUNK,), i32),
        pltpu.VMEM((CHUNK,), i32),
        pltpu.VMEM((CHUNK, D), f32),
        pltpu.VMEM((CHUNK,), f32),
        pltpu.SemaphoreType.DMA,
        pltpu.SemaphoreType.DMA,
    ],
    compiler_params=pltpu.CompilerParams(use_tc_tiling_on_sc=False),
)
def _gather_kernel(h4f, pdt, srcf, dstg, rows_out, pdg_out,
                   sidx, didx, rbuf, pbuf, sem_r, sem_p):
    wid = lax.axis_index("s") * 2 + lax.axis_index("c")
    per_tile = EGP // 32

    def chunk(k, carry):
        base = wid * per_tile + k * CHUNK
        pltpu.sync_copy(srcf.at[pl.ds(base, CHUNK)], sidx)
        pltpu.sync_copy(dstg.at[pl.ds(base, CHUNK)], didx)
        cps = []
        for j in range(NSTREAM):
            sl = pl.ds(j * 128, 128)
            cps.append(pltpu.async_copy(h4f.at[sidx.at[sl]], rbuf.at[sl], sem_r))
            cps.append(pltpu.async_copy(pdt.at[didx.at[sl]], pbuf.at[sl], sem_p))
        for cp in cps:
            cp.wait()
        pltpu.sync_copy(rbuf, rows_out.at[pl.ds(base, CHUNK)])
        pltpu.sync_copy(pbuf, pdg_out.at[pl.ds(base, CHUNK)])
        return carry

    lax.fori_loop(0, per_tile // CHUNK, chunk, 0)


def _scale_body(rows_ref, pdg_ref, as_ref, scaled_ref, w_ref):
    rows = rows_ref[...]
    ps = jnp.dot(rows, as_ref[0, 0].reshape(D, 1), preferred_element_type=f32,
                 precision=jax.lax.Precision.HIGHEST)
    e = ps[:, 0] + pdg_ref[...].reshape(-1)
    e = jnp.where(e > 0, e, 0.2 * e)
    w = jnp.exp(e)
    scaled_ref[...] = rows * w[:, None]
    w_ref[...] = w.reshape(w_ref.shape)


def _stage_c(rows, pdg2, a_src):
    bk = 4096
    nb = E0P // bk
    return pl.pallas_call(
        _scale_body,
        grid=(NL, nb),
        in_specs=[
            pl.BlockSpec((bk, D), lambda c, i: (c * nb + i, 0)),
            pl.BlockSpec((bk // 128, 128), lambda c, i: (c * nb + i, 0)),
            pl.BlockSpec((1, 1, D), lambda c, i: (c, 0, 0)),
        ],
        out_specs=[
            pl.BlockSpec((bk, D), lambda c, i: (c * nb + i, 0)),
            pl.BlockSpec((bk // 128, 128), lambda c, i: (c * nb + i, 0)),
        ],
        out_shape=[
            jax.ShapeDtypeStruct((EGP, D), f32),
            jax.ShapeDtypeStruct((EGP // 128, 128), f32),
        ],
    )(rows, pdg2, a_src.reshape(NL, 1, D))


@functools.partial(
    pl.kernel,
    out_type=(
        jax.ShapeDtypeStruct((NL, N, D), f32),
        jax.ShapeDtypeStruct((NL, NDEN), f32),
    ),
    mesh=_mesh,
    scratch_types=[
        pltpu.VMEM_SHARED((ACC_ROWS, D), f32),
        pltpu.VMEM_SHARED((ACC_ROWS,), f32),
        pltpu.VMEM((DCHUNK, D), f32),
        pltpu.VMEM((DCHUNK,), f32),
        pltpu.VMEM((DCHUNK,), i32),
        pltpu.VMEM((DNS, 128), i32),
        pltpu.SemaphoreType.DMA,
    ],
    compiler_params=pltpu.CompilerParams(use_tc_tiling_on_sc=False),
)
def _scatter_kernel(scaledf, wf, dstn, zrow, zden, acc_out, den_out,
                    shacc, shden, sbuf, wbuf, dbuf, idx2, sem):
    cid = lax.axis_index("c")
    sid = lax.axis_index("s")
    off = cid * HALF
    per_tile = E0P // 16

    for c in range(NL):
        def zchunk(kk, carry):
            ch = kk * 16 + sid

            @pl.when(ch < HALF // ZCH)
            def _():
                pltpu.sync_copy(zrow, shacc.at[pl.ds(ch * ZCH, ZCH)])
                pltpu.sync_copy(zden, shden.at[pl.ds(ch * ZCH, ZCH)])
            return carry

        lax.fori_loop(0, 2, zchunk, 0)

        @pl.when(sid == 0)
        def _():
            pltpu.sync_copy(zrow.at[pl.ds(0, 8)], shacc.at[pl.ds(HALF, 8)])
            pltpu.sync_copy(zden.at[pl.ds(0, 8)], shden.at[pl.ds(HALF, 8)])
        plsc.subcore_barrier()

        def echunk(k, carry):
            base = c * E0P + sid * per_tile + k * DCHUNK
            l1 = pltpu.async_copy(dstn.at[pl.ds(base, DCHUNK)], dbuf, sem)
            l2 = pltpu.async_copy(wf.at[pl.ds(base, DCHUNK)], wbuf, sem)
            l3 = pltpu.async_copy(scaledf.at[pl.ds(base, DCHUNK)], sbuf, sem)
            l1.wait()
            l2.wait()
            l3.wait()
            for j in range(DNS):
                for q in range(8):
                    dv = dbuf[pl.ds(j * 128 + q * 16, 16)]
                    lc = dv - off
                    ok = (lc >= 0) & (lc < HALF)
                    idx2[j, pl.ds(q * 16, 16)] = jnp.where(ok, lc, TRASH)
            for j in range(DNS):
                sl = pl.ds(j * 128, 128)
                pltpu.sync_copy(sbuf.at[sl], shacc.at[idx2.at[j]], add=True)
                pltpu.sync_copy(wbuf.at[sl], shden.at[idx2.at[j]], add=True)
            return carry

        lax.fori_loop(0, per_tile // DCHUNK, echunk, 0)
        plsc.subcore_barrier()

        def ochunk(kk, carry):
            ch = kk * 16 + sid

            @pl.when(ch < HALF // ZCH)
            def _():
                row = off + ch * ZCH
                pltpu.sync_copy(shacc.at[pl.ds(ch * ZCH, ZCH)],
                                acc_out.at[c, pl.ds(row, ZCH)])
                pltpu.sync_copy(shden.at[pl.ds(ch * ZCH, ZCH)],
                                den_out.at[c, pl.ds(row, ZCH)])
            return carry

        lax.fori_loop(0, 2, ochunk, 0)
        plsc.subcore_barrier()


def _mlp_body(x_ref, h4_ref, pd_ref, acc_ref, den_ref, as_ref, bagg_ref,
              w1_ref, b1_ref, w2_ref, b2_ref, y_ref):
    xb = x_ref[...]
    t = jnp.dot(xb, w1_ref[0:D, :], preferred_element_type=f32)
    for c in range(NL):
        hb = h4_ref[c]
        ps = jnp.sum(hb * as_ref[c][None, :], axis=1)
        e = ps + pd_ref[:, c]
        e = jnp.where(e > 0, e, 0.2 * e)
        wl = jnp.exp(e)
        den = den_ref[c].reshape(-1)
        inv = (1.0 / (den + wl + 1e-16))[:, None]
        out = (acc_ref[c] + wl[:, None] * hb) * inv + bagg_ref[c][None, :]
        t = t + jnp.dot(out, w1_ref[D * (c + 1):D * (c + 2), :],
                        preferred_element_type=f32)
    t = jnp.tanh(t + b1_ref[0][None, :])
    y_ref[...] = jnp.dot(t, w2_ref[...], preferred_element_type=f32) + b2_ref[0][None, :]


def _stage_e(x, h4, pd4, acc, den, a_src, b_agg, w1, b1, w2, b2):
    bk = 2048
    nb = (N + bk - 1) // bk
    den3 = den.reshape(NL, NDEN // 128, 128)
    return pl.pallas_call(
        _mlp_body,
        grid=(nb,),
        in_specs=[
            pl.BlockSpec((bk, D), lambda i: (i, 0)),
            pl.BlockSpec((NL, bk, D), lambda i: (0, i, 0)),
            pl.BlockSpec((bk, NL), lambda i: (i, 0)),
            pl.BlockSpec((NL, bk, D), lambda i: (0, i, 0)),
            pl.BlockSpec((NL, bk // 128, 128), lambda i: (0, i, 0)),
            pl.BlockSpec((NL, D), lambda i: (0, 0)),
            pl.BlockSpec((NL, D), lambda i: (0, 0)),
            pl.BlockSpec(((NL + 1) * D, D), lambda i: (0, 0)),
            pl.BlockSpec((1, D), lambda i: (0, 0)),
            pl.BlockSpec((D, D), lambda i: (0, 0)),
            pl.BlockSpec((1, D), lambda i: (0, 0)),
        ],
        out_specs=pl.BlockSpec((bk, D), lambda i: (i, 0)),
        out_shape=jax.ShapeDtypeStruct((N, D), f32),
    )(x, h4, pd4, acc, den3, a_src, b_agg, w1,
      b1.reshape(1, D), w2, b2.reshape(1, D))


def kernel(x, edge_index, W_agg, a_src, a_dst, b_agg, W1, b1, W2, b2):
    src = edge_index[0].reshape(NL, E0).astype(i32)
    dst = edge_index[1].reshape(NL, E0).astype(i32)
    offs = (jnp.arange(NL, dtype=i32) * N)[:, None]
    srcf = jnp.concatenate([src + offs, jnp.zeros((NL, PAD), i32)], 1).reshape(-1)
    dstg = jnp.concatenate([dst * NL + jnp.arange(NL, dtype=i32)[:, None],
                            jnp.zeros((NL, PAD), i32)], 1).reshape(-1)
    dstn = jnp.concatenate([dst, jnp.full((NL, PAD), N, i32)], 1).reshape(-1)
    zrow = jnp.zeros((ZCH, D), f32)
    zden = jnp.zeros((ZCH,), f32)

    h = x
    for l in range(L):
        h4, pd4 = _stage_a(h, W_agg[l], a_dst[l])
        rows, pdg = _gather_kernel(h4.reshape(NL * N, D), pd4.reshape(-1), srcf, dstg)
        scaled, w2d = _stage_c(rows, pdg.reshape(EGP // 128, 128), a_src[l])
        acc, den = _scatter_kernel(scaled, w2d.reshape(-1), dstn, zrow, zden)
        h = _stage_e(h, h4, pd4, acc, den, a_src[l], b_agg[l],
                     W1[l], b1[l], W2[l], b2[l])
    return h


# speedup vs baseline: 1.0353x; 1.0075x over previous
import functools

import jax
import jax.numpy as jnp
from jax import lax
from jax.experimental import pallas as pl
from jax.experimental.pallas import tpu as pltpu
from jax.experimental.pallas import tpu_sc as plsc

N = 50000
D = 64
NL = 4
L = 2
E0 = 200000
PAD = 704
E0P = E0 + PAD
EGP = NL * E0P
CHUNK = 896
NSTREAM = CHUNK // 128
DCHUNK = 128
HALF = 25000
TRASH = HALF
ACC_ROWS = HALF + 8
ZCH = 1000
NP = 51200
NDEN = 57344

_mesh = plsc.VectorSubcoreMesh(core_axis_name="c", subcore_axis_name="s")
f32 = jnp.float32
i32 = jnp.int32


def _proj_body(x_ref, w_ref, ad_ref, h4_ref, pd_ref):
    xb = x_ref[...]
    cols = []
    for c in range(NL):
        h = jnp.dot(xb, w_ref[c], preferred_element_type=f32)
        h4_ref[c] = h
        cols.append(jnp.sum(h * ad_ref[c][None, :], axis=1))
    pd_ref[...] = jnp.stack(cols, axis=1)


def _stage_a(x, w_agg, a_dst):
    bk = 2000
    return pl.pallas_call(
        _proj_body,
        grid=(N // bk,),
        in_specs=[
            pl.BlockSpec((bk, D), lambda i: (i, 0)),
            pl.BlockSpec((NL, D, D), lambda i: (0, 0, 0)),
            pl.BlockSpec((NL, D), lambda i: (0, 0)),
        ],
        out_specs=[
            pl.BlockSpec((NL, bk, D), lambda i: (0, i, 0)),
            pl.BlockSpec((bk, NL), lambda i: (i, 0)),
        ],
        out_shape=[
            jax.ShapeDtypeStruct((NL, N, D), f32),
            jax.ShapeDtypeStruct((N, NL), f32),
        ],
    )(x, w_agg, a_dst)


@functools.partial(
    pl.kernel,
    out_type=(
        jax.ShapeDtypeStruct((EGP, D), f32),
        jax.ShapeDtypeStruct((EGP,), f32),
    ),
    mesh=_mesh,
    scratch_types=[
        pltpu.VMEM((CHUNK,), i32),
        pltpu.VMEM((CHUNK,), i32),
        pltpu.VMEM((CHUNK, D), f32),
        pltpu.VMEM((CHUNK,), f32),
        pltpu.SemaphoreType.DMA,
        pltpu.SemaphoreType.DMA,
    ],
    compiler_params=pltpu.CompilerParams(use_tc_tiling_on_sc=False),
)
def _gather_kernel(h4f, pdt, srcf, dstg, rows_out, pdg_out,
                   sidx, didx, rbuf, pbuf, sem_r, sem_p):
    wid = lax.axis_index("s") * 2 + lax.axis_index("c")
    per_tile = EGP // 32

    def chunk(k, carry):
        base = wid * per_tile + k * CHUNK
        pltpu.sync_copy(srcf.at[pl.ds(base, CHUNK)], sidx)
        pltpu.sync_copy(dstg.at[pl.ds(base, CHUNK)], didx)
        cps = []
        for j in range(NSTREAM):
            sl = pl.ds(j * 128, 128)
            cps.append(pltpu.async_copy(h4f.at[sidx.at[sl]], rbuf.at[sl], sem_r))
            cps.append(pltpu.async_copy(pdt.at[didx.at[sl]], pbuf.at[sl], sem_p))
        for cp in cps:
            cp.wait()
        pltpu.sync_copy(rbuf, rows_out.at[pl.ds(base, CHUNK)])
        pltpu.sync_copy(pbuf, pdg_out.at[pl.ds(base, CHUNK)])
        return carry

    lax.fori_loop(0, per_tile // CHUNK, chunk, 0)


def _scale_body(rows_ref, pdg_ref, as_ref, scaled_ref, w_ref):
    rows = rows_ref[...]
    ps = jnp.dot(rows, as_ref[0, 0].reshape(D, 1), preferred_element_type=f32,
                 precision=jax.lax.Precision.HIGHEST)
    e = ps[:, 0] + pdg_ref[...].reshape(-1)
    e = jnp.where(e > 0, e, 0.2 * e)
    w = jnp.exp(e)
    scaled_ref[...] = rows * w[:, None]
    w_ref[...] = w.reshape(w_ref.shape)


def _stage_c(rows, pdg2, a_src):
    bk = 4096
    nb = E0P // bk
    return pl.pallas_call(
        _scale_body,
        grid=(NL, nb),
        in_specs=[
            pl.BlockSpec((bk, D), lambda c, i: (c * nb + i, 0)),
            pl.BlockSpec((bk // 128, 128), lambda c, i: (c * nb + i, 0)),
            pl.BlockSpec((1, 1, D), lambda c, i: (c, 0, 0)),
        ],
        out_specs=[
            pl.BlockSpec((bk, D), lambda c, i: (c * nb + i, 0)),
            pl.BlockSpec((bk // 128, 128), lambda c, i: (c * nb + i, 0)),
        ],
        out_shape=[
            jax.ShapeDtypeStruct((EGP, D), f32),
            jax.ShapeDtypeStruct((EGP // 128, 128), f32),
        ],
    )(rows, pdg2, a_src.reshape(NL, 1, D))


@functools.partial(
    pl.kernel,
    out_type=(
        jax.ShapeDtypeStruct((NL, N, D), f32),
        jax.ShapeDtypeStruct((NL, NDEN), f32),
    ),
    mesh=_mesh,
    scratch_types=[
        pltpu.VMEM_SHARED((ACC_ROWS, D), f32),
        pltpu.VMEM_SHARED((ACC_ROWS,), f32),
        pltpu.VMEM((2, DCHUNK, D), f32),
        pltpu.VMEM((2, DCHUNK), f32),
        pltpu.VMEM((2, DCHUNK), i32),
        pltpu.VMEM((2, DCHUNK), i32),
        pltpu.SemaphoreType.DMA,
        pltpu.SemaphoreType.DMA,
    ],
    compiler_params=pltpu.CompilerParams(use_tc_tiling_on_sc=False),
)
def _scatter_kernel(scaledf, wf, dstn, zrow, zden, acc_out, den_out,
                    shacc, shden, sbuf, wbuf, dbuf, idx2, semA, semB):
    cid = lax.axis_index("c")
    sid = lax.axis_index("s")
    off = cid * HALF
    per_tile = E0P // 16

    def fire(base, p, sem):
        h1 = pltpu.async_copy(dstn.at[pl.ds(base, DCHUNK)], dbuf.at[p], sem)
        h2 = pltpu.async_copy(wf.at[pl.ds(base, DCHUNK)], wbuf.at[p], sem)
        h3 = pltpu.async_copy(scaledf.at[pl.ds(base, DCHUNK)], sbuf.at[p], sem)
        return h1, h2, h3

    def drain(base, p, sem):
        pltpu.make_async_copy(dstn.at[pl.ds(base, DCHUNK)], dbuf.at[p], sem).wait()
        pltpu.make_async_copy(wf.at[pl.ds(base, DCHUNK)], wbuf.at[p], sem).wait()
        pltpu.make_async_copy(scaledf.at[pl.ds(base, DCHUNK)], sbuf.at[p], sem).wait()

    def consume(p):
        for q in range(DCHUNK // 16):
            dv = dbuf[p, pl.ds(q * 16, 16)]
            lc = dv - off
            ok = (lc >= 0) & (lc < HALF)
            idx2[p, pl.ds(q * 16, 16)] = jnp.where(ok, lc, TRASH)
        pltpu.sync_copy(sbuf.at[p], shacc.at[idx2.at[p]], add=True)
        pltpu.sync_copy(wbuf.at[p], shden.at[idx2.at[p]], add=True)

    for c in range(NL):
        def zchunk(kk, carry):
            ch = kk * 16 + sid

            @pl.when(ch < HALF // ZCH)
            def _():
                pltpu.sync_copy(zrow, shacc.at[pl.ds(ch * ZCH, ZCH)])
                pltpu.sync_copy(zden, shden.at[pl.ds(ch * ZCH, ZCH)])
            return carry

        lax.fori_loop(0, 2, zchunk, 0)

        @pl.when(sid == 0)
        def _():
            pltpu.sync_copy(zrow.at[pl.ds(0, 8)], shacc.at[pl.ds(HALF, 8)])
            pltpu.sync_copy(zden.at[pl.ds(0, 8)], shden.at[pl.ds(HALF, 8)])
        plsc.subcore_barrier()

        cbase = c * E0P + sid * per_tile
        fire(cbase, 0, semA)

        def echunk(k2, carry):
            k = 2 * k2
            fire(cbase + (k + 1) * DCHUNK, 1, semB)
            drain(cbase + k * DCHUNK, 0, semA)
            consume(0)

            @pl.when(k2 < per_tile // DCHUNK // 2 - 1)
            def _():
                fire(cbase + (k + 2) * DCHUNK, 0, semA)
            drain(cbase + (k + 1) * DCHUNK, 1, semB)
            consume(1)
            return carry

        lax.fori_loop(0, per_tile // DCHUNK // 2, echunk, 0)
        plsc.subcore_barrier()

        def ochunk(kk, carry):
            ch = kk * 16 + sid

            @pl.when(ch < HALF // ZCH)
            def _():
                row = off + ch * ZCH
                pltpu.sync_copy(shacc.at[pl.ds(ch * ZCH, ZCH)],
                                acc_out.at[c, pl.ds(row, ZCH)])
                pltpu.sync_copy(shden.at[pl.ds(ch * ZCH, ZCH)],
                                den_out.at[c, pl.ds(row, ZCH)])
            return carry

        lax.fori_loop(0, 2, ochunk, 0)
        plsc.subcore_barrier()


def _mlp_body(x_ref, h4_ref, pd_ref, acc_ref, den_ref, as_ref, bagg_ref,
              w1_ref, b1_ref, w2_ref, b2_ref, y_ref):
    xb = x_ref[...]
    t = jnp.dot(xb, w1_ref[0:D, :], preferred_element_type=f32)
    for c in range(NL):
        hb = h4_ref[c]
        ps = jnp.sum(hb * as_ref[c][None, :], axis=1)
        e = ps + pd_ref[:, c]
        e = jnp.where(e > 0, e, 0.2 * e)
        wl = jnp.exp(e)
        den = den_ref[c].reshape(-1)
        inv = (1.0 / (den + wl + 1e-16))[:, None]
        out = (acc_ref[c] + wl[:, None] * hb) * inv + bagg_ref[c][None, :]
        t = t + jnp.dot(out, w1_ref[D * (c + 1):D * (c + 2), :],
                        preferred_element_type=f32)
    t = jnp.tanh(t + b1_ref[0][None, :])
    y_ref[...] = jnp.dot(t, w2_ref[...], preferred_element_type=f32) + b2_ref[0][None, :]


def _stage_e(x, h4, pd4, acc, den, a_src, b_agg, w1, b1, w2, b2):
    bk = 2048
    nb = (N + bk - 1) // bk
    den3 = den.reshape(NL, NDEN // 128, 128)
    return pl.pallas_call(
        _mlp_body,
        grid=(nb,),
        in_specs=[
            pl.BlockSpec((bk, D), lambda i: (i, 0)),
            pl.BlockSpec((NL, bk, D), lambda i: (0, i, 0)),
            pl.BlockSpec((bk, NL), lambda i: (i, 0)),
            pl.BlockSpec((NL, bk, D), lambda i: (0, i, 0)),
            pl.BlockSpec((NL, bk // 128, 128), lambda i: (0, i, 0)),
            pl.BlockSpec((NL, D), lambda i: (0, 0)),
            pl.BlockSpec((NL, D), lambda i: (0, 0)),
            pl.BlockSpec(((NL + 1) * D, D), lambda i: (0, 0)),
            pl.BlockSpec((1, D), lambda i: (0, 0)),
            pl.BlockSpec((D, D), lambda i: (0, 0)),
            pl.BlockSpec((1, D), lambda i: (0, 0)),
        ],
        out_specs=pl.BlockSpec((bk, D), lambda i: (i, 0)),
        out_shape=jax.ShapeDtypeStruct((N, D), f32),
    )(x, h4, pd4, acc, den3, a_src, b_agg, w1,
      b1.reshape(1, D), w2, b2.reshape(1, D))


def kernel(x, edge_index, W_agg, a_src, a_dst, b_agg, W1, b1, W2, b2):
    src = edge_index[0].reshape(NL, E0).astype(i32)
    dst = edge_index[1].reshape(NL, E0).astype(i32)
    offs = (jnp.arange(NL, dtype=i32) * N)[:, None]
    srcf = jnp.concatenate([src + offs, jnp.zeros((NL, PAD), i32)], 1).reshape(-1)
    dstg = jnp.concatenate([dst * NL + jnp.arange(NL, dtype=i32)[:, None],
                            jnp.zeros((NL, PAD), i32)], 1).reshape(-1)
    dstn = jnp.concatenate([dst, jnp.full((NL, PAD), N, i32)], 1).reshape(-1)
    zrow = jnp.zeros((ZCH, D), f32)
    zden = jnp.zeros((ZCH,), f32)

    h = x
    for l in range(L):
        h4, pd4 = _stage_a(h, W_agg[l], a_dst[l])
        rows, pdg = _gather_kernel(h4.reshape(NL * N, D), pd4.reshape(-1), srcf, dstg)
        scaled, w2d = _stage_c(rows, pdg.reshape(EGP // 128, 128), a_src[l])
        acc, den = _scatter_kernel(scaled, w2d.reshape(-1), dstn, zrow, zden)
        h = _stage_e(h, h4, pd4, acc, den, a_src[l], b_agg[l],
                     W1[l], b1[l], W2[l], b2[l])
    return h


# speedup vs baseline: 1.0360x; 1.0007x over previous
import functools

import jax
import jax.numpy as jnp
from jax import lax
from jax.experimental import pallas as pl
from jax.experimental.pallas import tpu as pltpu
from jax.experimental.pallas import tpu_sc as plsc

N = 50000
D = 64
NL = 4
L = 2
E0 = 200000
PAD = 704
E0P = E0 + PAD
EGP = NL * E0P
CHUNK = 896
NSTREAM = CHUNK // 128
DCHUNK = 128
HALF = 25000
TRASH = HALF
ACC_ROWS = HALF + 8
ZCH = 1000
NDEN = 57344

_mesh = plsc.VectorSubcoreMesh(core_axis_name="c", subcore_axis_name="s")
f32 = jnp.float32
i32 = jnp.int32


def _proj_body(x_ref, w_ref, ad_ref, h4_ref, pd_ref):
    xb = x_ref[...]
    cols = []
    for c in range(NL):
        h = jnp.dot(xb, w_ref[c], preferred_element_type=f32)
        h4_ref[c] = h
        cols.append(jnp.sum(h * ad_ref[c][None, :], axis=1))
    pd_ref[...] = jnp.stack(cols, axis=1)


def _stage_a(x, w_agg, a_dst):
    bk = 2000
    return pl.pallas_call(
        _proj_body,
        grid=(N // bk,),
        in_specs=[
            pl.BlockSpec((bk, D), lambda i: (i, 0)),
            pl.BlockSpec((NL, D, D), lambda i: (0, 0, 0)),
            pl.BlockSpec((NL, D), lambda i: (0, 0)),
        ],
        out_specs=[
            pl.BlockSpec((NL, bk, D), lambda i: (0, i, 0)),
            pl.BlockSpec((bk, NL), lambda i: (i, 0)),
        ],
        out_shape=[
            jax.ShapeDtypeStruct((NL, N, D), f32),
            jax.ShapeDtypeStruct((N, NL), f32),
        ],
    )(x, w_agg, a_dst)


@functools.partial(
    pl.kernel,
    out_type=(
        jax.ShapeDtypeStruct((EGP, D), f32),
        jax.ShapeDtypeStruct((EGP,), f32),
    ),
    mesh=_mesh,
    scratch_types=[
        pltpu.VMEM((CHUNK,), i32),
        pltpu.VMEM((CHUNK,), i32),
        pltpu.VMEM((CHUNK, D), f32),
        pltpu.VMEM((CHUNK,), f32),
        pltpu.SemaphoreType.DMA,
        pltpu.SemaphoreType.DMA,
    ],
    compiler_params=pltpu.CompilerParams(use_tc_tiling_on_sc=False),
)
def _gather_kernel(h4f, pdt, srcf, dstg, rows_out, pdg_out,
                   sidx, didx, rbuf, pbuf, sem_r, sem_p):
    wid = lax.axis_index("s") * 2 + lax.axis_index("c")
    per_tile = EGP // 32

    def chunk(k, carry):
        base = wid * per_tile + k * CHUNK
        pltpu.sync_copy(srcf.at[pl.ds(base, CHUNK)], sidx)
        pltpu.sync_copy(dstg.at[pl.ds(base, CHUNK)], didx)
        cps = []
        for j in range(NSTREAM):
            sl = pl.ds(j * 128, 128)
            cps.append(pltpu.async_copy(h4f.at[sidx.at[sl]], rbuf.at[sl], sem_r))
            cps.append(pltpu.async_copy(pdt.at[didx.at[sl]], pbuf.at[sl], sem_p))
        for cp in cps:
            cp.wait()
        pltpu.sync_copy(rbuf, rows_out.at[pl.ds(base, CHUNK)])
        pltpu.sync_copy(pbuf, pdg_out.at[pl.ds(base, CHUNK)])
        return carry

    lax.fori_loop(0, per_tile // CHUNK, chunk, 0)


def _scale_body(rows_ref, pdg_ref, as_ref, scaled_ref, w_ref):
    rows = rows_ref[...]
    ps = jnp.dot(rows, as_ref[0, 0].reshape(D, 1), preferred_element_type=f32,
                 precision=jax.lax.Precision.HIGHEST)
    e = ps[:, 0] + pdg_ref[...].reshape(-1)
    e = jnp.where(e > 0, e, 0.2 * e)
    w = jnp.exp(e)
    scaled_ref[...] = rows * w[:, None]
    w_ref[...] = w.reshape(w_ref.shape)


def _stage_c(rows, pdg2, a_src):
    bk = 4096
    nb = E0P // bk
    return pl.pallas_call(
        _scale_body,
        grid=(NL, nb),
        in_specs=[
            pl.BlockSpec((bk, D), lambda c, i: (c * nb + i, 0)),
            pl.BlockSpec((bk // 128, 128), lambda c, i: (c * nb + i, 0)),
            pl.BlockSpec((1, 1, D), lambda c, i: (c, 0, 0)),
        ],
        out_specs=[
            pl.BlockSpec((bk, D), lambda c, i: (c * nb + i, 0)),
            pl.BlockSpec((bk // 128, 128), lambda c, i: (c * nb + i, 0)),
        ],
        out_shape=[
            jax.ShapeDtypeStruct((EGP, D), f32),
            jax.ShapeDtypeStruct((EGP // 128, 128), f32),
        ],
    )(rows, pdg2, a_src.reshape(NL, 1, D))


@functools.partial(
    pl.kernel,
    out_type=(
        jax.ShapeDtypeStruct((NL, N, D), f32),
        jax.ShapeDtypeStruct((NL, NDEN), f32),
    ),
    mesh=_mesh,
    scratch_types=[
        pltpu.VMEM_SHARED((ACC_ROWS, D), f32),
        pltpu.VMEM_SHARED((ACC_ROWS,), f32),
        pltpu.VMEM((2, DCHUNK, D), f32),
        pltpu.VMEM((2, DCHUNK), f32),
        pltpu.VMEM((2, DCHUNK), i32),
        pltpu.VMEM((2, DCHUNK), i32),
        pltpu.SemaphoreType.DMA,
        pltpu.SemaphoreType.DMA,
    ],
    compiler_params=pltpu.CompilerParams(use_tc_tiling_on_sc=False),
)
def _scatter_kernel(scaledf, wf, dstn, zrow, zden, acc_out, den_out,
                    shacc, shden, sbuf, wbuf, dbuf, idx2, semA, semB):
    cid = lax.axis_index("c")
    sid = lax.axis_index("s")
    off = cid * HALF
    per_tile = E0P // 16

    def fire(base, p, sem):
        h1 = pltpu.async_copy(dstn.at[pl.ds(base, DCHUNK)], dbuf.at[p], sem)
        h2 = pltpu.async_copy(wf.at[pl.ds(base, DCHUNK)], wbuf.at[p], sem)
        h3 = pltpu.async_copy(scaledf.at[pl.ds(base, DCHUNK)], sbuf.at[p], sem)
        return h1, h2, h3

    def drain(base, p, sem):
        pltpu.make_async_copy(dstn.at[pl.ds(base, DCHUNK)], dbuf.at[p], sem).wait()
        pltpu.make_async_copy(wf.at[pl.ds(base, DCHUNK)], wbuf.at[p], sem).wait()
        pltpu.make_async_copy(scaledf.at[pl.ds(base, DCHUNK)], sbuf.at[p], sem).wait()

    def consume(p):
        for q in range(DCHUNK // 16):
            dv = dbuf[p, pl.ds(q * 16, 16)]
            lc = dv - off
            ok = (lc >= 0) & (lc < HALF)
            idx2[p, pl.ds(q * 16, 16)] = jnp.where(ok, lc, TRASH)
        pltpu.sync_copy(sbuf.at[p], shacc.at[idx2.at[p]], add=True)
        pltpu.sync_copy(wbuf.at[p], shden.at[idx2.at[p]], add=True)

    for c in range(NL):
        def zchunk(kk, carry):
            ch = kk * 16 + sid

            @pl.when(ch < HALF // ZCH)
            def _():
                pltpu.sync_copy(zrow, shacc.at[pl.ds(ch * ZCH, ZCH)])
                pltpu.sync_copy(zden, shden.at[pl.ds(ch * ZCH, ZCH)])
            return carry

        lax.fori_loop(0, 2, zchunk, 0)

        @pl.when(sid == 0)
        def _():
            pltpu.sync_copy(zrow.at[pl.ds(0, 8)], shacc.at[pl.ds(HALF, 8)])
            pltpu.sync_copy(zden.at[pl.ds(0, 8)], shden.at[pl.ds(HALF, 8)])
        plsc.subcore_barrier()

        cbase = c * E0P + sid * per_tile
        fire(cbase, 0, semA)

        def echunk(k2, carry):
            k = 2 * k2
            fire(cbase + (k + 1) * DCHUNK, 1, semB)
            drain(cbase + k * DCHUNK, 0, semA)
            consume(0)

            @pl.when(k2 < per_tile // DCHUNK // 2 - 1)
            def _():
                fire(cbase + (k + 2) * DCHUNK, 0, semA)
            drain(cbase + (k + 1) * DCHUNK, 1, semB)
            consume(1)
            return carry

        lax.fori_loop(0, per_tile // DCHUNK // 2, echunk, 0)
        plsc.subcore_barrier()

        def ochunk(kk, carry):
            ch = kk * 16 + sid

            @pl.when(ch < HALF // ZCH)
            def _():
                row = off + ch * ZCH
                pltpu.sync_copy(shacc.at[pl.ds(ch * ZCH, ZCH)],
                                acc_out.at[c, pl.ds(row, ZCH)])
                pltpu.sync_copy(shden.at[pl.ds(ch * ZCH, ZCH)],
                                den_out.at[c, pl.ds(row, ZCH)])
            return carry

        lax.fori_loop(0, 2, ochunk, 0)
        plsc.subcore_barrier()


def _mlp_body(x_ref, h4_ref, pd_ref, acc_ref, den_ref, as_ref, bagg_ref,
              w1_ref, b1_ref, w2_ref, b2_ref, y_ref):
    xb = x_ref[...]
    t = jnp.dot(xb, w1_ref[0:D, :], preferred_element_type=f32)
    for c in range(NL):
        hb = h4_ref[c]
        ps = jnp.sum(hb * as_ref[c][None, :], axis=1)
        e = ps + pd_ref[:, c]
        e = jnp.where(e > 0, e, 0.2 * e)
        wl = jnp.exp(e)
        den = den_ref[c].reshape(-1)
        inv = (1.0 / (den + wl + 1e-16))[:, None]
        out = (acc_ref[c] + wl[:, None] * hb) * inv + bagg_ref[c][None, :]
        t = t + jnp.dot(out, w1_ref[D * (c + 1):D * (c + 2), :],
                        preferred_element_type=f32)
    t = jnp.tanh(t + b1_ref[0][None, :])
    y_ref[...] = jnp.dot(t, w2_ref[...], preferred_element_type=f32) + b2_ref[0][None, :]


def _stage_e(x, h4, pd4, acc, den, a_src, b_agg, w1, b1, w2, b2):
    bk = 2048
    nb = (N + bk - 1) // bk
    den3 = den.reshape(NL, NDEN // 128, 128)
    return pl.pallas_call(
        _mlp_body,
        grid=(nb,),
        in_specs=[
            pl.BlockSpec((bk, D), lambda i: (i, 0)),
            pl.BlockSpec((NL, bk, D), lambda i: (0, i, 0)),
            pl.BlockSpec((bk, NL), lambda i: (i, 0)),
            pl.BlockSpec((NL, bk, D), lambda i: (0, i, 0)),
            pl.BlockSpec((NL, bk // 128, 128), lambda i: (0, i, 0)),
            pl.BlockSpec((NL, D), lambda i: (0, 0)),
            pl.BlockSpec((NL, D), lambda i: (0, 0)),
            pl.BlockSpec(((NL + 1) * D, D), lambda i: (0, 0)),
            pl.BlockSpec((1, D), lambda i: (0, 0)),
            pl.BlockSpec((D, D), lambda i: (0, 0)),
            pl.BlockSpec((1, D), lambda i: (0, 0)),
        ],
        out_specs=pl.BlockSpec((bk, D), lambda i: (i, 0)),
        out_shape=jax.ShapeDtypeStruct((N, D), f32),
    )(x, h4, pd4, acc, den3, a_src, b_agg, w1,
      b1.reshape(1, D), w2, b2.reshape(1, D))


def kernel(x, edge_index, W_agg, a_src, a_dst, b_agg, W1, b1, W2, b2):
    src = edge_index[0].reshape(NL, E0).astype(i32)
    dst = edge_index[1].reshape(NL, E0).astype(i32)
    offs = (jnp.arange(NL, dtype=i32) * N)[:, None]
    srcf = jnp.concatenate([src + offs, jnp.zeros((NL, PAD), i32)], 1).reshape(-1)
    dstg = jnp.concatenate([dst * NL + jnp.arange(NL, dtype=i32)[:, None],
                            jnp.zeros((NL, PAD), i32)], 1).reshape(-1)
    dstn = jnp.concatenate([dst, jnp.full((NL, PAD), N, i32)], 1).reshape(-1)
    zrow = jnp.zeros((ZCH, D), f32)
    zden = jnp.zeros((ZCH,), f32)

    h = x
    for l in range(L):
        h4, pd4 = _stage_a(h, W_agg[l], a_dst[l])
        rows, pdg = _gather_kernel(h4.reshape(NL * N, D), pd4.reshape(-1), srcf, dstg)
        scaled, w2d = _stage_c(rows, pdg.reshape(EGP // 128, 128), a_src[l])
        acc, den = _scatter_kernel(scaled, w2d.reshape(-1), dstn, zrow, zden)
        h = _stage_e(h, h4, pd4, acc, den, a_src[l], b_agg[l],
                     W1[l], b1[l], W2[l], b2[l])
    return h
